# initial kernel scaffold (unmeasured)
import jax
import jax.numpy as jnp
from jax import lax
from jax.experimental import pallas as pl
from jax.experimental.pallas import tpu as pltpu

N_DEV = 16
SQ = 256
SKV_PER = 4096
HQ = 8
DH = 128
D_MODEL = 1024
BLK = 64
SCALE = 0.08838834764831843
DISTS = (1, 2, 4, 8)


def _body(x_ref, wq_ref, k_ref, v_ref, wo_ref, out_ref,
          o_acc, l_acc, o_recv, l_recv,
          o_send_sems, o_recv_sems, l_send_sems, l_recv_sems):
    my = lax.axis_index("i")

    q_all = jnp.dot(x_ref[...], wq_ref[...], preferred_element_type=jnp.float32)
    qb = lax.broadcasted_iota(jnp.int32, (SQ, SKV_PER), 0) // BLK
    kb = (lax.broadcasted_iota(jnp.int32, (SQ, SKV_PER), 1) // BLK
          + my * (SKV_PER // BLK))
    mask = (qb == kb) | (kb == 0) | (((qb + kb) % 3) == 0)
    ones_row = jnp.ones((1, SKV_PER), jnp.float32)
    for h in range(HQ):
        qh = q_all[:, h * DH:(h + 1) * DH]
        kh = k_ref[:, h * DH:(h + 1) * DH]
        vh = v_ref[:, h * DH:(h + 1) * DH]
        s = lax.dot_general(qh, kh, (((1,), (1,)), ((), ())),
                            preferred_element_type=jnp.float32) * SCALE
        e = jnp.where(mask, jnp.exp(s), 0.0)
        o_acc[h, :, :] = lax.dot_general(e, vh, (((1,), (0,)), ((), ())),
                                         preferred_element_type=jnp.float32)
        l_acc[h:h + 1, :] = lax.dot_general(
            ones_row, e, (((1,), (1,)), ((), ())),
            preferred_element_type=jnp.float32)

    barrier = pltpu.get_barrier_semaphore()
    for d in DISTS:
        pl.semaphore_signal(barrier, inc=1, device_id=(my ^ d,),
                            device_id_type=pl.DeviceIdType.MESH)
    pl.semaphore_wait(barrier, len(DISTS))

    for r, d in enumerate(DISTS):
        partner = my ^ d
        o_rdma = pltpu.make_async_remote_copy(
            src_ref=o_acc, dst_ref=o_recv.at[r],
            send_sem=o_send_sems.at[r], recv_sem=o_recv_sems.at[r],
            device_id=(partner,), device_id_type=pl.DeviceIdType.MESH)
        l_rdma = pltpu.make_async_remote_copy(
            src_ref=l_acc, dst_ref=l_recv.at[r],
            send_sem=l_send_sems.at[r], recv_sem=l_recv_sems.at[r],
            device_id=(partner,), device_id_type=pl.DeviceIdType.MESH)
        o_rdma.start()
        l_rdma.start()
        o_rdma.wait()
        l_rdma.wait()
        o_acc[...] = o_acc[...] + o_recv[r]
        l_acc[...] = l_acc[...] + l_recv[r]

    ones_col = jnp.ones((1, DH), jnp.float32)
    out = jnp.zeros((SQ, D_MODEL), jnp.float32)
    for h in range(HQ):
        lh = lax.dot_general(l_acc[h:h + 1, :], ones_col,
                             (((0,), (0,)), ((), ())),
                             preferred_element_type=jnp.float32)
        ctx = o_acc[h, :, :] / lh
        out = out + jnp.dot(ctx, wo_ref[h * DH:(h + 1) * DH, :],
                            preferred_element_type=jnp.float32)
    out_ref[...] = out


def kernel(x, Wq, K_ext, V_ext, Wo):
    x2 = x.reshape(SQ, D_MODEL)
    k2 = K_ext.reshape(SKV_PER, HQ * DH)
    v2 = V_ext.reshape(SKV_PER, HQ * DH)
    out = pl.pallas_call(
        _body,
        out_shape=jax.ShapeDtypeStruct((SQ, D_MODEL), jnp.float32),
        in_specs=[pl.BlockSpec(memory_space=pltpu.VMEM)] * 5,
        out_specs=pl.BlockSpec(memory_space=pltpu.VMEM),
        scratch_shapes=[
            pltpu.VMEM((HQ, SQ, DH), jnp.float32),
            pltpu.VMEM((HQ, SQ), jnp.float32),
            pltpu.VMEM((len(DISTS), HQ, SQ, DH), jnp.float32),
            pltpu.VMEM((len(DISTS), HQ, SQ), jnp.float32),
            pltpu.SemaphoreType.DMA((len(DISTS),)),
            pltpu.SemaphoreType.DMA((len(DISTS),)),
            pltpu.SemaphoreType.DMA((len(DISTS),)),
            pltpu.SemaphoreType.DMA((len(DISTS),)),
        ],
        compiler_params=pltpu.CompilerParams(collective_id=0),
    )(x2, Wq, k2, v2, Wo)
    return out.reshape(1, SQ, D_MODEL)


# baseline (device time: 131280 ns/iter reference)
import jax
import jax.numpy as jnp
from jax import lax
from jax.experimental import pallas as pl
from jax.experimental.pallas import tpu as pltpu

N_DEV = 16
SQ = 256
SKV_PER = 4096
HQ = 8
DH = 128
D_MODEL = 1024
BLK = 64
SCALE = 0.08838834764831843
DISTS = (1, 2, 4, 8)


def _body(x_ref, wq_ref, k_ref, v_ref, wo_ref, out_ref,
          o_acc, l_acc, o_recv, l_recv,
          o_send_sems, o_recv_sems, l_send_sems, l_recv_sems):
    my = lax.axis_index("i")

    q_all = jnp.dot(x_ref[...], wq_ref[...], preferred_element_type=jnp.float32)
    qb = lax.broadcasted_iota(jnp.int32, (SQ, SKV_PER), 0) // BLK
    kb = (lax.broadcasted_iota(jnp.int32, (SQ, SKV_PER), 1) // BLK
          + my * (SKV_PER // BLK))
    mask = (qb == kb) | (kb == 0) | (((qb + kb) % 3) == 0)
    ones_row = jnp.ones((1, SKV_PER), jnp.float32)
    for h in range(HQ):
        qh = q_all[:, h * DH:(h + 1) * DH]
        kh = k_ref[:, h * DH:(h + 1) * DH]
        vh = v_ref[:, h * DH:(h + 1) * DH]
        s = lax.dot_general(qh, kh, (((1,), (1,)), ((), ())),
                            preferred_element_type=jnp.float32) * SCALE
        e = jnp.where(mask, jnp.exp(s), 0.0)
        o_acc[h, :, :] = lax.dot_general(e, vh, (((1,), (0,)), ((), ())),
                                         preferred_element_type=jnp.float32)
        l_acc[h:h + 1, :] = lax.dot_general(
            ones_row, e, (((1,), (1,)), ((), ())),
            preferred_element_type=jnp.float32)

    barrier = pltpu.get_barrier_semaphore()
    for d in DISTS:
        pl.semaphore_signal(barrier, inc=1, device_id=(my ^ d,),
                            device_id_type=pl.DeviceIdType.MESH)
    pl.semaphore_wait(barrier, len(DISTS))

    for r, d in enumerate(DISTS):
        partner = my ^ d
        o_rdma = pltpu.make_async_remote_copy(
            src_ref=o_acc, dst_ref=o_recv.at[r],
            send_sem=o_send_sems.at[r], recv_sem=o_recv_sems.at[r],
            device_id=(partner,), device_id_type=pl.DeviceIdType.MESH)
        l_rdma = pltpu.make_async_remote_copy(
            src_ref=l_acc, dst_ref=l_recv.at[r],
            send_sem=l_send_sems.at[r], recv_sem=l_recv_sems.at[r],
            device_id=(partner,), device_id_type=pl.DeviceIdType.MESH)
        o_rdma.start()
        l_rdma.start()
        o_rdma.wait()
        l_rdma.wait()
        o_acc[...] = o_acc[...] + o_recv[r]
        l_acc[...] = l_acc[...] + l_recv[r]

    ones_col = jnp.ones((1, DH), jnp.float32)
    out = jnp.zeros((SQ, D_MODEL), jnp.float32)
    for h in range(HQ):
        lh = lax.dot_general(l_acc[h:h + 1, :], ones_col,
                             (((0,), (0,)), ((), ())),
                             preferred_element_type=jnp.float32)
        ctx = o_acc[h, :, :] / lh
        out = out + jnp.dot(ctx, wo_ref[h * DH:(h + 1) * DH, :],
                            preferred_element_type=jnp.float32)
    out_ref[...] = out


def kernel(x, Wq, K_ext, V_ext, Wo):
    x2 = x.reshape(SQ, D_MODEL)
    k2 = K_ext.reshape(SKV_PER, HQ * DH)
    v2 = V_ext.reshape(SKV_PER, HQ * DH)
    out = pl.pallas_call(
        _body,
        out_shape=jax.ShapeDtypeStruct((SQ, D_MODEL), jnp.float32),
        in_specs=[pl.BlockSpec(memory_space=pltpu.VMEM)] * 5,
        out_specs=pl.BlockSpec(memory_space=pltpu.VMEM),
        scratch_shapes=[
            pltpu.VMEM((HQ, SQ, DH), jnp.float32),
            pltpu.VMEM((HQ, SQ), jnp.float32),
            pltpu.VMEM((len(DISTS), HQ, SQ, DH), jnp.float32),
            pltpu.VMEM((len(DISTS), HQ, SQ), jnp.float32),
            pltpu.SemaphoreType.DMA((len(DISTS),)),
            pltpu.SemaphoreType.DMA((len(DISTS),)),
            pltpu.SemaphoreType.DMA((len(DISTS),)),
            pltpu.SemaphoreType.DMA((len(DISTS),)),
        ],
        compiler_params=pltpu.CompilerParams(
            collective_id=0, vmem_limit_bytes=100 * 1024 * 1024),
    )(x2, Wq, k2, v2, Wo)
    return out.reshape(1, SQ, D_MODEL)
